# baseline (speedup 1.0000x reference)
import jax
import jax.numpy as jnp
import numpy as np
from jax import lax
from jax.experimental import pallas as pl
from jax.experimental.pallas import tpu as pltpu
from jax.experimental.pallas import tpu_sc as plsc

_N_NODE = 1000000
_V = 1000000
_B = 16384
_R = 4
_D = 32

_NW = 32
_TPW = 244
_MAIN = _TPW * _NW * 128
_TAIL_LO = _MAIN
_TAILW = _V - _MAIN
_PT = 4
_PW = _PT * 128
_NP = _TPW // _PT
_IDXC = 8192
_CAP_HT = 2048
_CAP_REL = 6144
_CAP_CUR = 1024

_NBAT = 1280
_HT_BASE = _B * _R
_SENT = _HT_BASE + _B
_NROWS = _SENT + 16


def _smax(v):
    return jnp.max(v)


def _k1_body(ht_t, rel_t, ht_tail, rel_tail, idx_ht_h, idx_rel_h, r_in,
             rows_st, slots_st, tots,
             idxchunk, mi_ht, ms_ht, mi_rel, ms_rel,
             pbuf_ht0, pbuf_ht1, pbuf_rel0, pbuf_rel1,
             tbuf, cur_i, cur_s, rowbuf, sbuf4, v16,
             sem_in, sem_p0, sem_p1, sem_o0, sem_o1, sem_o2, sem_o3):
    w = lax.axis_index("s") * 2 + lax.axis_index("c")
    t0 = w * _TPW
    lo = t0 * 128
    hi = lo + _TPW * 128
    iota = lax.iota(jnp.int32, 16)

    pltpu.async_copy(r_in, v16, sem_in).wait()
    r = _smax(v16[...])

    def scan_list(idx_h, n, slot_base, mi, ms, rlo, rhi):
        def big_step(bg, carry):
            tot, pos = carry
            pltpu.async_copy(idx_h.at[pl.ds(bg * _IDXC, _IDXC)], idxchunk,
                             sem_in).wait()

            def chunk(i, c2):
                tot, pos = c2
                v = idxchunk[pl.ds(i * 16, 16)]
                m = jnp.logical_or(
                    jnp.logical_and(v >= lo, v < hi), v >= _TAIL_LO)
                mi32 = m.astype(jnp.int32)
                rank = tot + plsc.cumsum(mi32) - 1
                sel = jnp.logical_and(
                    m, jnp.logical_and(rank >= rlo, rank < rhi))
                plsc.store_compressed(mi.at[pl.ds(pos, 16)], v, mask=sel)
                slots = slot_base + bg * _IDXC + i * 16 + iota
                plsc.store_compressed(ms.at[pl.ds(pos, 16)], slots, mask=sel)
                return (tot + jnp.sum(mi32),
                        pos + jnp.sum(sel.astype(jnp.int32)))

            return lax.fori_loop(0, _IDXC // 16, chunk, (tot, pos))

        return lax.fori_loop(0, n // _IDXC, big_step,
                             (np.int32(0), np.int32(0)))

    def extract(pbuf, width, cb, mi, ms, nstored, crlo, crhi, bcount):
        def mchunk(g, c2):
            ptot, ppos = c2
            mask_valid = (g * 16 + iota) < nstored
            c16 = mi[pl.ds(g * 16, 16)]
            s16 = ms[pl.ds(g * 16, 16)]
            inp = jnp.logical_and(
                mask_valid,
                jnp.logical_and(c16 >= cb, c16 < cb + width))
            prank = ptot + plsc.cumsum(inp.astype(jnp.int32)) - 1
            sel = jnp.logical_and(
                inp, jnp.logical_and(prank >= crlo, prank < crhi))
            plsc.store_compressed(cur_i.at[pl.ds(ppos, 16)], c16 - cb,
                                  mask=sel)
            plsc.store_compressed(cur_s.at[pl.ds(ppos, 16)], s16, mask=sel)
            return (ptot + jnp.sum(inp.astype(jnp.int32)),
                    ppos + jnp.sum(sel.astype(jnp.int32)))

        nmc = (nstored + 15) // 16
        _ptot, pcount = lax.fori_loop(0, nmc, mchunk,
                                      (np.int32(0), np.int32(0)))

        sem_o = [sem_o0, sem_o1, sem_o2, sem_o3]

        def group(g, bc):
            for b in range(4):
                gg = g * 4 + b
                bcg = bc + gg

                @pl.when(bcg >= 4)
                def _(b=b):
                    pltpu.make_async_copy(
                        rowbuf.at[pl.ds(b * 512, 512)],
                        rows_st.at[pl.ds(0, 512)], sem_o[b]).wait()
                    pltpu.make_async_copy(
                        sbuf4.at[b], slots_st.at[pl.ds(0, 16)],
                        sem_o[b]).wait()

                rem = jnp.minimum(pcount - gg * 16, 16)
                cvec = jnp.where(iota < rem, cur_i[pl.ds(gg * 16, 16)], 0)
                svec = jnp.where(iota < rem, cur_s[pl.ds(gg * 16, 16)],
                                 _SENT)
                sbuf4[b, :] = svec
                for d in range(_D):
                    dv = jnp.full((16,), d, jnp.int32)
                    vals = plsc.load_gather(pbuf, [dv, cvec])
                    plsc.store_scatter(rowbuf, [(b * 16 + iota) * _D + dv],
                                       vals)
                pltpu.async_copy(
                    rowbuf.at[pl.ds(b * 512, 512)],
                    rows_st.at[pl.ds((w * _NBAT + bcg) * 512, 512)],
                    sem_o[b])
                pltpu.async_copy(
                    sbuf4.at[b],
                    slots_st.at[pl.ds((w * _NBAT + bcg) * 16, 16)],
                    sem_o[b])
            return bc

        ngrp = (pcount + 63) // 64
        lax.fori_loop(0, ngrp, group, bcount)
        return bcount + ngrp * 4

    def count_members(width, cb, mi, nstored):
        def cchunk(g, ptot):
            mask_valid = (g * 16 + iota) < nstored
            c16 = mi[pl.ds(g * 16, 16)]
            inp = jnp.logical_and(
                mask_valid,
                jnp.logical_and(c16 >= cb, c16 < cb + width))
            return ptot + jnp.sum(inp.astype(jnp.int32))

        return lax.fori_loop(0, (nstored + 15) // 16, cchunk, np.int32(0))

    def extract_windowed(pbuf, width, cb, mi, ms, nstored, bcount):
        ptot = count_members(width, cb, mi, nstored)

        def body(rr, bc):
            return extract(pbuf, width, cb, mi, ms, nstored,
                           rr * _CAP_CUR, (rr + 1) * _CAP_CUR, bc)

        return lax.fori_loop(0, (ptot + _CAP_CUR - 1) // _CAP_CUR, body,
                             bcount)

    def issue_piece(p, pbh, pbr, semp):
        cb = (t0 + p * _PT) * 128
        for tr in range(4):
            pltpu.async_copy(ht_t.at[pl.ds(tr * 8, 8), pl.ds(cb, _PW)],
                             pbh.at[pl.ds(tr * 8, 8), :], semp)
            pltpu.async_copy(rel_t.at[pl.ds(tr * 8, 8), pl.ds(cb, _PW)],
                             pbr.at[pl.ds(tr * 8, 8), :], semp)

    def drain_piece(pbh, pbr, semp):
        for tr in range(4):
            pltpu.make_async_copy(ht_t.at[pl.ds(0, 8), pl.ds(0, _PW)],
                                  pbh.at[pl.ds(tr * 8, 8), :], semp).wait()
            pltpu.make_async_copy(rel_t.at[pl.ds(0, 8), pl.ds(0, _PW)],
                                  pbr.at[pl.ds(tr * 8, 8), :], semp).wait()

    issue_piece(0, pbuf_ht0, pbuf_rel0, sem_p0)
    tot_ht, n_ht = scan_list(idx_ht_h, _B, _HT_BASE, mi_ht, ms_ht,
                             r * _CAP_HT, (r + 1) * _CAP_HT)
    tot_rel, n_rel = scan_list(idx_rel_h, _B * _R, 0, mi_rel, ms_rel,
                               r * _CAP_REL, (r + 1) * _CAP_REL)

    def use(pbh, pbr, cb, bc1):
        bc1 = extract_windowed(pbh, _PW, cb, mi_ht, ms_ht, n_ht, bc1)
        bc1 = extract_windowed(pbr, _PW, cb, mi_rel, ms_rel, n_rel, bc1)
        return bc1

    issue_piece(1, pbuf_ht1, pbuf_rel1, sem_p1)

    def piece_step(j, bc):
        p0 = j * 2
        drain_piece(pbuf_ht0, pbuf_rel0, sem_p0)
        bc = use(pbuf_ht0, pbuf_rel0, (t0 + p0 * _PT) * 128, bc)
        issue_piece(p0 + 2, pbuf_ht0, pbuf_rel0, sem_p0)
        drain_piece(pbuf_ht1, pbuf_rel1, sem_p1)
        bc = use(pbuf_ht1, pbuf_rel1, (t0 + (p0 + 1) * _PT) * 128, bc)

        @pl.when(p0 + 3 < _NP)
        def _():
            issue_piece(p0 + 3, pbuf_ht1, pbuf_rel1, sem_p1)
        return bc

    bcount = lax.fori_loop(0, (_NP - 1) // 2, piece_step, np.int32(0))
    drain_piece(pbuf_ht0, pbuf_rel0, sem_p0)
    bcount = use(pbuf_ht0, pbuf_rel0, (t0 + (_NP - 1) * _PT) * 128, bcount)

    pltpu.async_copy(ht_tail, tbuf, sem_in).wait()
    bcount = extract_windowed(tbuf, _TAILW, _TAIL_LO, mi_ht, ms_ht, n_ht,
                              bcount)
    pltpu.async_copy(rel_tail, tbuf, sem_in).wait()
    bcount = extract_windowed(tbuf, _TAILW, _TAIL_LO, mi_rel, ms_rel, n_rel,
                              bcount)

    v16[...] = jnp.where(
        iota == 0, tot_ht,
        jnp.where(iota == 1, tot_rel, jnp.where(iota == 2, bcount, 0)))
    pltpu.async_copy(v16, tots.at[pl.ds(w * 16, 16)], sem_in).wait()

    for b, sem_b in enumerate((sem_o0, sem_o1, sem_o2, sem_o3)):
        @pl.when(bcount > b)
        def _(b=b, sem_b=sem_b):
            pltpu.make_async_copy(rowbuf.at[pl.ds(b * 512, 512)],
                                  rows_st.at[pl.ds(0, 512)], sem_b).wait()
            pltpu.make_async_copy(sbuf4.at[b], slots_st.at[pl.ds(0, 16)],
                                  sem_b).wait()


def _k1(ht_t, rel_t, ht_tail, rel_tail, idx_ht, idx_rel, r_arr):
    mesh = plsc.VectorSubcoreMesh(core_axis_name="c", subcore_axis_name="s")
    fn = pl.kernel(
        _k1_body,
        mesh=mesh,
        out_type=(
            jax.ShapeDtypeStruct((_NW * _NBAT * 16 * _D,), jnp.float32),
            jax.ShapeDtypeStruct((_NW * _NBAT * 16,), jnp.int32),
            jax.ShapeDtypeStruct((_NW * 16,), jnp.int32),
        ),
        scratch_types=[
            pltpu.VMEM((_IDXC,), jnp.int32),
            pltpu.VMEM((_CAP_HT + 16,), jnp.int32),
            pltpu.VMEM((_CAP_HT + 16,), jnp.int32),
            pltpu.VMEM((_CAP_REL + 16,), jnp.int32),
            pltpu.VMEM((_CAP_REL + 16,), jnp.int32),
            pltpu.VMEM((_D, _PW), jnp.float32),
            pltpu.VMEM((_D, _PW), jnp.float32),
            pltpu.VMEM((_D, _PW), jnp.float32),
            pltpu.VMEM((_D, _PW), jnp.float32),
            pltpu.VMEM((_D, _TAILW), jnp.float32),
            pltpu.VMEM((_CAP_CUR + 16,), jnp.int32),
            pltpu.VMEM((_CAP_CUR + 16,), jnp.int32),
            pltpu.VMEM((4 * 16 * _D,), jnp.float32),
            pltpu.VMEM((4, 16), jnp.int32),
            pltpu.VMEM((16,), jnp.int32),
            pltpu.SemaphoreType.DMA,
            pltpu.SemaphoreType.DMA,
            pltpu.SemaphoreType.DMA,
            pltpu.SemaphoreType.DMA,
            pltpu.SemaphoreType.DMA,
            pltpu.SemaphoreType.DMA,
            pltpu.SemaphoreType.DMA,
        ],
        compiler_params=pltpu.CompilerParams(
            use_tc_tiling_on_sc=True, needs_layout_passes=False),
    )
    return fn(ht_t, rel_t, ht_tail, rel_tail, idx_ht, idx_rel, r_arr)


def _k2_body(rows_st, slots_st, tots, prev, r_in, out,
             rows_v, slots_v, s2d, cbuf, pv, v16, sem_in,
             sem_s0, sem_s1, sem_s2, sem_s3):
    w = lax.axis_index("s") * 2 + lax.axis_index("c")
    iota = lax.iota(jnp.int32, 16)
    pltpu.async_copy(r_in, v16, sem_in).wait()
    r = _smax(v16[...])

    @pl.when(r > 0)
    def _():
        base = w * (_SENT // _NW)

        def cp(i, _c):
            pltpu.async_copy(prev.at[pl.ds(base + i * 512, 512), :], pv,
                             sem_in).wait()
            pltpu.async_copy(pv, out.at[pl.ds(base + i * 512, 512), :],
                             sem_in).wait()
            return _c
        lax.fori_loop(0, _SENT // _NW // 512, cp, 0)

    pltpu.async_copy(tots.at[pl.ds(w * 16, 16)], v16, sem_in).wait()
    bcount = jnp.sum(jnp.where(iota == 2, v16[...], 0))

    nb16_tot_worker = bcount

    def chunk_step(c, bc):
        sbase = w * (_NBAT * 16) + c * 64 * 16
        pltpu.async_copy(slots_st.at[pl.ds(sbase, 1024)], slots_v,
                         sem_in).wait()
        pltpu.async_copy(rows_st.at[pl.ds(sbase * _D, 1024 * _D)], rows_v,
                         sem_in).wait()

        sem_s = [sem_s0, sem_s1, sem_s2, sem_s3]

        def g128(g, bc2):
            for b in range(4):
                sb = bc2 + g * 4 + b
                k0 = (g * 4 + b) * 8

                @pl.when(c * 64 + k0 < nb16_tot_worker)
                def _(b=b, sb=sb, k0=k0):
                    @pl.when(sb >= 4)
                    def _():
                        pltpu.make_async_copy(cbuf.at[b],
                                              out.at[s2d.at[b]],
                                              sem_s[b]).wait()

                    for j in range(8):
                        real = (c * 64 + k0 + j) < nb16_tot_worker
                        s16 = jnp.where(
                            real, slots_v[pl.ds((k0 + j) * 16, 16)], _SENT)
                        s2d[b, pl.ds(j * 16, 16)] = s16
                    for j in range(256):
                        cbuf[b, j // 2, pl.ds((j % 2) * 16, 16)] = (
                            rows_v[pl.ds(k0 * 16 * _D + j * 16, 16)])
                    pltpu.async_copy(cbuf.at[b], out.at[s2d.at[b]],
                                     sem_s[b])
            return bc2

        lax.fori_loop(0, 2, g128, bc)
        return bc + 8

    bc_fin = lax.fori_loop(0, (bcount + 63) // 64, chunk_step, np.int32(0))
    nsb = (bcount + 7) // 8

    for b, sem_b in enumerate((sem_s0, sem_s1, sem_s2, sem_s3)):
        @pl.when(nsb > b)
        def _(b=b, sem_b=sem_b):
            pltpu.make_async_copy(cbuf.at[b], out.at[s2d.at[b]],
                                  sem_b).wait()


def _k2(rows_st, slots_st, tots, prev, r_arr):
    mesh = plsc.VectorSubcoreMesh(core_axis_name="c", subcore_axis_name="s")
    fn = pl.kernel(
        _k2_body,
        mesh=mesh,
        out_type=jax.ShapeDtypeStruct((_NROWS, _D), jnp.float32),
        scratch_types=[
            pltpu.VMEM((1024 * _D,), jnp.float32),
            pltpu.VMEM((1024,), jnp.int32),
            pltpu.VMEM((4, 128), jnp.int32),
            pltpu.VMEM((4, 128, _D), jnp.float32),
            pltpu.VMEM((512, _D), jnp.float32),
            pltpu.VMEM((16,), jnp.int32),
            pltpu.SemaphoreType.DMA,
            pltpu.SemaphoreType.DMA,
            pltpu.SemaphoreType.DMA,
            pltpu.SemaphoreType.DMA,
            pltpu.SemaphoreType.DMA,
        ],
        compiler_params=pltpu.CompilerParams(
            use_tc_tiling_on_sc=False, needs_layout_passes=False),
    )
    return fn(rows_st, slots_st, tots, prev, r_arr)


def _tc_loss_body(ht_ref, rel_ref, gt_ref, out_ref):
    i = pl.program_id(0)
    ht = ht_ref[...]
    s_ht = jnp.sum(ht * ht, axis=-1, keepdims=True)
    inv_ht = 1.0 / jnp.maximum(jnp.sqrt(s_ht), 1e-12)
    total = jnp.zeros((1, 1), jnp.float32)
    for r in range(_R):
        rel = rel_ref[:, r * _D:(r + 1) * _D]
        dot = jnp.sum(rel * ht, axis=-1, keepdims=True)
        ss = jnp.sum(rel * rel, axis=-1, keepdims=True)
        inv_rel = 1.0 / jnp.maximum(jnp.sqrt(ss), 1e-12)
        score = dot * inv_rel * inv_ht
        z = -score * gt_ref[:, r:r + 1]
        loss = jnp.maximum(z, 0.0) + jnp.log1p(jnp.exp(-jnp.abs(z)))
        total = total + jnp.sum(loss, axis=0, keepdims=True)

    @pl.when(i == 0)
    def _():
        out_ref[...] = jnp.zeros((1, 1), jnp.float32)

    out_ref[...] += total * (1.0 / (_B * _R))


def _tc_loss(ht_rows, rel_rows, gt):
    nb = 1024
    grid = _B // nb
    return pl.pallas_call(
        _tc_loss_body,
        grid=(grid,),
        in_specs=[
            pl.BlockSpec((nb, _D), lambda i: (i, 0)),
            pl.BlockSpec((nb, _R * _D), lambda i: (i, 0)),
            pl.BlockSpec((nb, _R), lambda i: (i, 0)),
        ],
        out_specs=pl.BlockSpec((1, 1), lambda i: (0, 0)),
        out_shape=jax.ShapeDtypeStruct((1, 1), jnp.float32),
    )(ht_rows, rel_rows, gt)


def kernel(hyper_node_embeddings, base, base_edge_index, ground_truth, rel_table):
    idx_ht = (jnp.reshape(base_edge_index, (_B,)) - _N_NODE).astype(jnp.int32)
    idx_rel = jnp.reshape(base, (_B * _R,)).astype(jnp.int32)
    ht_t = jnp.transpose(hyper_node_embeddings)
    rel_t = jnp.transpose(rel_table)
    ht_tail = ht_t[:, _TAIL_LO:]
    rel_tail = rel_t[:, _TAIL_LO:]

    def body(c):
        r, out, mh, mr = c
        r_arr = jnp.full((16,), r, jnp.int32)
        rows_st, slots_st, tots = _k1(ht_t, rel_t, ht_tail, rel_tail,
                                      idx_ht, idx_rel, r_arr)
        out2 = _k2(rows_st, slots_st, tots, out, r_arr)
        t = tots.reshape(_NW, 16)
        return (r + 1, out2, jnp.max(t[:, 0]), jnp.max(t[:, 1]))

    def cond(c):
        r, out, mh, mr = c
        return jnp.logical_or(
            r == 0,
            jnp.logical_or(mh > r * _CAP_HT, mr > r * _CAP_REL))

    init = (jnp.int32(0), jnp.zeros((_NROWS, _D), jnp.float32),
            jnp.int32(2 ** 30), jnp.int32(2 ** 30))
    _, allrows, _, _ = lax.while_loop(cond, body, init)

    rel_rows = allrows[0:_B * _R].reshape(_B, _R * _D)
    ht_rows = allrows[_HT_BASE:_HT_BASE + _B]
    out = _tc_loss(ht_rows, rel_rows, ground_truth)
    return out[0, 0]


# speedup vs baseline: 2.4788x; 2.4788x over previous
import jax
import jax.numpy as jnp
from jax import lax
from jax.experimental import pallas as pl
from jax.experimental.pallas import tpu as pltpu
from jax.experimental.pallas import tpu_sc as plsc

_N_NODE = 1000000
_B = 16384
_R = 4
_D = 32

_NC = 2
_NS = 16
_NW = _NC * _NS
_BPW = _B // _NW


def _sc_gather_body(ht_tab, rel_tab, idx_ht, idx_rel, ht_out, rel_out,
                    idx_ht_v, idx_rel_v, ht_v, rel_v, sem1, sem2):
    wid = lax.axis_index("s") * _NC + lax.axis_index("c")
    base = wid * _BPW
    pltpu.sync_copy(idx_ht.at[pl.ds(base, _BPW)], idx_ht_v)
    pltpu.sync_copy(idx_rel.at[pl.ds(base * _R, _BPW * _R)], idx_rel_v)
    cp1 = pltpu.async_copy(ht_tab.at[idx_ht_v], ht_v, sem1)
    cp2 = pltpu.async_copy(rel_tab.at[idx_rel_v], rel_v, sem2)
    cp1.wait()
    cp2.wait()
    pltpu.sync_copy(ht_v, ht_out.at[pl.ds(base, _BPW)])
    pltpu.sync_copy(rel_v, rel_out.at[pl.ds(base * _R, _BPW * _R)])


def _sc_gather(ht_tab, rel_tab, idx_ht, idx_rel):
    mesh = plsc.VectorSubcoreMesh(core_axis_name="c", subcore_axis_name="s")
    fn = pl.kernel(
        _sc_gather_body,
        mesh=mesh,
        out_type=(
            jax.ShapeDtypeStruct((_B, _D), jnp.float32),
            jax.ShapeDtypeStruct((_B * _R, _D), jnp.float32),
        ),
        scratch_types=[
            pltpu.VMEM((_BPW,), jnp.int32),
            pltpu.VMEM((_BPW * _R,), jnp.int32),
            pltpu.VMEM((_BPW, _D), jnp.float32),
            pltpu.VMEM((_BPW * _R, _D), jnp.float32),
            pltpu.SemaphoreType.DMA,
            pltpu.SemaphoreType.DMA,
        ],
        compiler_params=pltpu.CompilerParams(use_tc_tiling_on_sc=False),
    )
    return fn(ht_tab, rel_tab, idx_ht, idx_rel)


def _tc_loss_body(ht_ref, rel_ref, gt_ref, out_ref):
    i = pl.program_id(0)
    ht = ht_ref[...]
    s_ht = jnp.sum(ht * ht, axis=-1, keepdims=True)
    inv_ht = 1.0 / jnp.maximum(jnp.sqrt(s_ht), 1e-12)
    total = jnp.zeros((1, 1), jnp.float32)
    for r in range(_R):
        rel = rel_ref[:, r * _D:(r + 1) * _D]
        dot = jnp.sum(rel * ht, axis=-1, keepdims=True)
        ss = jnp.sum(rel * rel, axis=-1, keepdims=True)
        inv_rel = 1.0 / jnp.maximum(jnp.sqrt(ss), 1e-12)
        score = dot * inv_rel * inv_ht
        z = -score * gt_ref[:, r:r + 1]
        loss = jnp.maximum(z, 0.0) + jnp.log1p(jnp.exp(-jnp.abs(z)))
        total = total + jnp.sum(loss, axis=0, keepdims=True)

    @pl.when(i == 0)
    def _():
        out_ref[...] = jnp.zeros((1, 1), jnp.float32)

    out_ref[...] += total * (1.0 / (_B * _R))


def _tc_loss(ht_rows, rel_rows, gt):
    nb = 1024
    grid = _B // nb
    return pl.pallas_call(
        _tc_loss_body,
        grid=(grid,),
        in_specs=[
            pl.BlockSpec((nb, _D), lambda i: (i, 0)),
            pl.BlockSpec((nb, _R * _D), lambda i: (i, 0)),
            pl.BlockSpec((nb, _R), lambda i: (i, 0)),
        ],
        out_specs=pl.BlockSpec((1, 1), lambda i: (0, 0)),
        out_shape=jax.ShapeDtypeStruct((1, 1), jnp.float32),
    )(ht_rows, rel_rows, gt)


def kernel(hyper_node_embeddings, base, base_edge_index, ground_truth, rel_table):
    idx_ht = (jnp.reshape(base_edge_index, (_B,)) - _N_NODE).astype(jnp.int32)
    idx_rel = jnp.reshape(base, (_B * _R,)).astype(jnp.int32)
    ht_rows, rel_rows = _sc_gather(hyper_node_embeddings, rel_table, idx_ht, idx_rel)
    out = _tc_loss(ht_rows, jnp.reshape(rel_rows, (_B, _R * _D)), ground_truth)
    return out[0, 0]
